# baseline (speedup 1.0000x reference)
import functools

import jax
import jax.numpy as jnp
from jax import lax
from jax.experimental import pallas as pl
from jax.experimental.pallas import tpu as pltpu
from jax.experimental.pallas import tpu_sc as plsc

EMBED_D = 316
B_TOTAL = 4096 * 20
GRAN = 32
GPR = 11
NUM_CORES = 2
NUM_SUBCORES = 16
NW = NUM_CORES * NUM_SUBCORES
B_PER_W = B_TOTAL // NW
CHUNK = 64
N_CHUNKS = B_PER_W // CHUNK
NIDX = CHUNK * GPR
SLOT = GPR * GRAN
PACK_W = CHUNK * EMBED_D

_mesh = plsc.VectorSubcoreMesh(core_axis_name="c", subcore_axis_name="s")


@functools.partial(
    pl.kernel,
    mesh=_mesh,
    out_type=jax.ShapeDtypeStruct((B_TOTAL * EMBED_D,), jnp.float32),
    scratch_types=[
        pltpu.VMEM((B_PER_W,), jnp.int32),
        pltpu.VMEM((NIDX,), jnp.int32),
        pltpu.VMEM((NIDX,), jnp.int32),
        pltpu.VMEM((NIDX, GRAN), jnp.float32),
        pltpu.VMEM((NIDX, GRAN), jnp.float32),
        pltpu.VMEM((PACK_W,), jnp.float32),
        pltpu.SemaphoreType.DMA,
        pltpu.SemaphoreType.DMA,
    ],
    compiler_params=pltpu.CompilerParams(
        use_tc_tiling_on_sc=False, needs_layout_passes=False
    ),
)
def _gather_kernel(idx_hbm, tview_hbm, out_hbm, idx_v, gidx0_v, gidx1_v,
                   fetch0_v, fetch1_v, packed_v, sem0, sem1):
    wid = lax.axis_index("s") * NUM_CORES + lax.axis_index("c")
    base = wid * B_PER_W
    gidxs = (gidx0_v, gidx1_v)
    fetches = (fetch0_v, fetch1_v)
    sems = (sem0, sem1)
    lanes = lax.iota(jnp.int32, 16)

    pltpu.sync_copy(idx_hbm.at[pl.ds(base, B_PER_W)], idx_v)

    def issue(c, slot):
        gidx = gidxs[slot]
        for g in range(CHUNK // 16):
            rowpos = g * 16 + lanes
            a = idx_v[pl.ds(c * CHUNK + g * 16, 16)]
            q = (a * EMBED_D) >> 5
            pos = rowpos * GPR
            for j in range(GPR):
                plsc.store_scatter(gidx, [pos + j], q + j)
        pltpu.async_copy(tview_hbm.at[gidx], fetches[slot], sems[slot])

    def process(c, slot):
        fetch = fetches[slot]
        pltpu.make_async_copy(tview_hbm.at[gidxs[slot]], fetch,
                              sems[slot]).wait()
        for g in range(CHUNK // 16):
            rowpos = g * 16 + lanes
            a = idx_v[pl.ds(c * CHUNK + g * 16, 16)]
            srcb = rowpos * SLOT + ((a * EMBED_D) & 31)
            dstb = rowpos * EMBED_D

            def body(w, carry):
                t = srcb + w
                v = plsc.load_gather(fetch, [t >> 5, t & 31])
                plsc.store_scatter(packed_v, [dstb + w], v)
                return carry

            lax.fori_loop(0, EMBED_D, body, 0, unroll=4)
        pltpu.sync_copy(
            packed_v,
            out_hbm.at[pl.ds(base * EMBED_D + c * PACK_W, PACK_W)],
        )

    issue(0, 0)

    def pair(g, carry):
        c = 2 * g

        @pl.when(c + 1 < N_CHUNKS)
        def _():
            issue(c + 1, 1)

        process(c, 0)

        @pl.when(c + 2 < N_CHUNKS)
        def _():
            issue(c + 2, 0)

        @pl.when(c + 1 < N_CHUNKS)
        def _():
            process(c + 1, 1)

        return carry

    lax.fori_loop(0, (N_CHUNKS + 1) // 2, pair, 0)


def kernel(indices, table):
    flat_idx = indices.reshape(-1)
    tview = table.reshape(-1, GRAN)
    out = _gather_kernel(flat_idx, tview)
    return out.reshape(indices.shape + (table.shape[1],))


# speedup vs baseline: 1.0767x; 1.0767x over previous
import functools

import jax
import jax.numpy as jnp
from jax import lax
from jax.experimental import pallas as pl
from jax.experimental.pallas import tpu as pltpu
from jax.experimental.pallas import tpu_sc as plsc

EMBED_D = 316
B_TOTAL = 4096 * 20
GRAN = 32
GPR = 11
NUM_CORES = 2
NUM_SUBCORES = 16
NW = NUM_CORES * NUM_SUBCORES
B_PER_W = B_TOTAL // NW
CHUNK = 64
N_CHUNKS = B_PER_W // CHUNK
NIDX = CHUNK * GPR
SLOT = GPR * GRAN
PACK_W = CHUNK * EMBED_D

_mesh = plsc.VectorSubcoreMesh(core_axis_name="c", subcore_axis_name="s")


@functools.partial(
    pl.kernel,
    mesh=_mesh,
    out_type=jax.ShapeDtypeStruct((B_TOTAL * EMBED_D,), jnp.float32),
    scratch_types=[
        pltpu.VMEM((B_PER_W,), jnp.int32),
        pltpu.VMEM((NIDX,), jnp.int32),
        pltpu.VMEM((NIDX,), jnp.int32),
        pltpu.VMEM((NIDX, GRAN), jnp.float32),
        pltpu.VMEM((NIDX, GRAN), jnp.float32),
        pltpu.VMEM((PACK_W + 8,), jnp.float32),
        pltpu.VMEM((PACK_W + 8,), jnp.float32),
        pltpu.SemaphoreType.DMA,
        pltpu.SemaphoreType.DMA,
        pltpu.SemaphoreType.DMA,
        pltpu.SemaphoreType.DMA,
    ],
    compiler_params=pltpu.CompilerParams(
        use_tc_tiling_on_sc=False, needs_layout_passes=False
    ),
)
def _gather_kernel(idx_hbm, tview_hbm, out_hbm, idx_v, gidx0_v, gidx1_v,
                   fetch0_v, fetch1_v, packed0_v, packed1_v,
                   sem0, sem1, wsem0, wsem1):
    wid = lax.axis_index("s") * NUM_CORES + lax.axis_index("c")
    base = wid * B_PER_W
    gidxs = (gidx0_v, gidx1_v)
    fetches = (fetch0_v, fetch1_v)
    packs = (packed0_v, packed1_v)
    sems = (sem0, sem1)
    wsems = (wsem0, wsem1)
    lanes = lax.iota(jnp.int32, 16)

    pltpu.sync_copy(idx_hbm.at[pl.ds(base, B_PER_W)], idx_v)

    def out_window(c, slot):
        return out_hbm.at[pl.ds(base * EMBED_D + c * PACK_W, PACK_W)]

    def issue(c, slot):
        gidx = gidxs[slot]
        for g in range(CHUNK // 16):
            a = idx_v[pl.ds(c * CHUNK + g * 16, 16)]
            q = (a * EMBED_D) >> 5
            pos = (g * 16 + lanes) * GPR
            for j in range(GPR):
                plsc.store_scatter(gidx, [pos + j], q + j)
        pltpu.async_copy(tview_hbm.at[gidx], fetches[slot], sems[slot])

    def process(c, slot):
        fetch = fetches[slot]
        packed = packs[slot]
        pltpu.make_async_copy(tview_hbm.at[gidxs[slot]], fetch,
                              sems[slot]).wait()

        @pl.when(c >= 2)
        def _():
            pltpu.make_async_copy(packed.at[pl.ds(0, PACK_W)],
                                  out_window(c, slot), wsems[slot]).wait()

        for g in range(CHUNK // 16):
            a = idx_v[pl.ds(c * CHUNK + g * 16, 16)]
            r16 = (a * EMBED_D) & 31

            def row(rl, carry):
                rli = g * 16 + rl
                rj = r16.at[lanes * 0 + rl].get(mode="promise_in_bounds")
                srcb = rj + rli * SLOT
                for k in range(EMBED_D // 16 + 1):
                    t = srcb + (k * 16 + lanes)
                    v = plsc.load_gather(fetch, [t >> 5, t & 31])
                    packed[pl.ds(rli * EMBED_D + k * 16, 16)] = v
                return carry

            lax.fori_loop(0, 16, row, 0)
        pltpu.async_copy(packed.at[pl.ds(0, PACK_W)], out_window(c, slot),
                         wsems[slot])

    issue(0, 0)

    def pair(g, carry):
        c = 2 * g

        @pl.when(c + 1 < N_CHUNKS)
        def _():
            issue(c + 1, 1)

        process(c, 0)

        @pl.when(c + 2 < N_CHUNKS)
        def _():
            issue(c + 2, 0)

        @pl.when(c + 1 < N_CHUNKS)
        def _():
            process(c + 1, 1)

        return carry

    lax.fori_loop(0, (N_CHUNKS + 1) // 2, pair, 0)

    pltpu.make_async_copy(packed0_v.at[pl.ds(0, PACK_W)],
                          out_window(N_CHUNKS - 2, 0), wsem0).wait()
    pltpu.make_async_copy(packed1_v.at[pl.ds(0, PACK_W)],
                          out_window(N_CHUNKS - 1, 1), wsem1).wait()


def kernel(indices, table):
    flat_idx = indices.reshape(-1)
    tview = table.reshape(-1, GRAN)
    out = _gather_kernel(flat_idx, tview)
    return out.reshape(indices.shape + (table.shape[1],))
